# Initial kernel scaffold; baseline (speedup 1.0000x reference)
import jax
import jax.numpy as jnp
from jax.experimental import pallas as pl


def kernel(x, pos, batch, lframes, W1, b1, W2, b2):
    raise NotImplementedError("write your pallas kernel here")


# speedup vs baseline: 8.0342x; 8.0342x over previous
import functools

import jax
import jax.numpy as jnp
from jax import lax
from jax.experimental import pallas as pl
from jax.experimental.pallas import tpu as pltpu

_B = 16
_INT_MAX = 2147483647


def _pick_chunk(n):
    for c in (2000, 1000, 500, 200, 100, 50, 8):
        if n % c == 0:
            return c
    return n


def _stats_body(pos_ref, batch_ref, out_ref):
    pid = pl.program_id(0)

    @pl.when(pid == 0)
    def _():
        out_ref[:] = jnp.zeros_like(out_ref)

    b = batch_ref[0, 0, :]
    seg = lax.broadcasted_iota(jnp.int32, (1, _B), 1)
    onehot = (b[:, None] == seg).astype(jnp.float32)
    ones = jnp.ones((b.shape[0], 1), dtype=jnp.float32)
    posext = jnp.concatenate([pos_ref[:], ones], axis=1)
    part = lax.dot_general(onehot, posext, (((0,), (0,)), ((), ())),
                           preferred_element_type=jnp.float32)
    out_ref[:] += part


def _argmin_body(pos_ref, batch_ref, stats_ref, md_ref, idx_ref):
    pid = pl.program_id(0)
    chunk = pos_ref.shape[0]

    @pl.when(pid == 0)
    def _():
        md_ref[:] = jnp.full_like(md_ref, jnp.inf)
        idx_ref[:] = jnp.full_like(idx_ref, _INT_MAX)

    cnt = stats_ref[:, 3:4]
    com = stats_ref[:, 0:3] / jnp.maximum(cnt, 1.0)
    b = batch_ref[0, 0, :]
    seg = lax.broadcasted_iota(jnp.int32, (1, _B), 1)
    onehot = (b[:, None] == seg)
    comb = jnp.dot(onehot.astype(jnp.float32), com,
                   preferred_element_type=jnp.float32)
    diff = pos_ref[:] - comb
    d = jnp.sqrt(jnp.sum(diff * diff, axis=1))
    dmask = jnp.where(onehot, d[:, None], jnp.inf)
    cmin = jnp.min(dmask, axis=0)
    gidx = pid * chunk + lax.broadcasted_iota(jnp.int32, (chunk, 1), 0)
    cand = jnp.where(onehot & (d[:, None] <= cmin[None, :]), gidx, _INT_MAX)
    cidx = jnp.min(cand, axis=0)
    better = cmin < md_ref[0, :]
    md_ref[0, :] = jnp.where(better, cmin, md_ref[0, :])
    idx_ref[0, :] = jnp.where(better, cidx, idx_ref[0, :])


def _main_body(nchunks, x_ref, pos_ref, batch_ref, xdst_ref, pdst_ref,
               l_ref, w1a_ref, w1b_ref, w1c_ref, b1_ref, w2_ref, b2_ref,
               out_ref, p_ref):
    pid = pl.program_id(0)
    chunk = x_ref.shape[0]
    H = w1a_ref.shape[1]

    @pl.when(pid == 0)
    def _():
        a_cols = []
        for j in range(3):
            aj = (l_ref[:, 0, j][:, None] * w1c_ref[0:1, :]
                  + l_ref[:, 1, j][:, None] * w1c_ref[1:2, :]
                  + l_ref[:, 2, j][:, None] * w1c_ref[2:3, :])
            p_ref[:, H * j:H * (j + 1)] = aj
            a_cols.append(aj)
        c = (jnp.dot(xdst_ref[:], w1b_ref[:],
                     preferred_element_type=jnp.float32)
             + b1_ref[:]
             - (pdst_ref[:, 0:1] * a_cols[0]
                + pdst_ref[:, 1:2] * a_cols[1]
                + pdst_ref[:, 2:3] * a_cols[2]))
        p_ref[:, H * 3:H * 4] = c
        out_ref[:] = jnp.full_like(out_ref, -jnp.inf)

    b = batch_ref[0, 0, :]
    seg = lax.broadcasted_iota(jnp.int32, (1, _B), 1)
    onehot = (b[:, None] == seg).astype(jnp.float32)
    g = jnp.dot(onehot, p_ref[:], preferred_element_type=jnp.float32)
    pre = jnp.dot(x_ref[:], w1a_ref[:], preferred_element_type=jnp.float32)
    pre = (pre
           + pos_ref[:, 0:1] * g[:, 0:H]
           + pos_ref[:, 1:2] * g[:, H:2 * H]
           + pos_ref[:, 2:3] * g[:, 2 * H:3 * H]
           + g[:, 3 * H:4 * H])
    h = jnp.maximum(pre, 0.0)
    msg = jnp.dot(h, w2_ref[:], preferred_element_type=jnp.float32)

    bf = b[0]
    bl = b[chunk - 1]

    @pl.when(bf == bl)
    def _():
        mx = jnp.max(msg, axis=0)
        cur = out_ref[pl.ds(bf, 1), :]
        out_ref[pl.ds(bf, 1), :] = jnp.maximum(cur, mx[None, :])

    @pl.when(bf != bl)
    def _():
        for s in range(_B):
            def _upd(s=s):
                mx = jnp.max(jnp.where(b[:, None] == s, msg, -jnp.inf),
                             axis=0)
                out_ref[s:s + 1, :] = jnp.maximum(out_ref[s:s + 1, :],
                                                  mx[None, :])
            pl.when((bf <= s) & (s <= bl))(_upd)

    @pl.when(pid == nchunks - 1)
    def _():
        out_ref[:] = out_ref[:] + b2_ref[:]


def kernel(x, pos, batch, lframes, W1, b1, W2, b2):
    n, d = x.shape
    h = W1.shape[1]
    chunk = _pick_chunk(n)
    nchunks = n // chunk
    batch3 = batch.reshape(nchunks, 1, chunk)
    f32 = jnp.float32

    row_spec = lambda width: pl.BlockSpec((chunk, width), lambda i: (i, 0))
    batch_spec = pl.BlockSpec((1, 1, chunk), lambda i: (i, 0, 0))
    full = lambda shape: pl.BlockSpec(shape, lambda i: tuple(0 for _ in shape))

    stats = pl.pallas_call(
        _stats_body,
        grid=(nchunks,),
        in_specs=[row_spec(3), batch_spec],
        out_specs=full((_B, 4)),
        out_shape=jax.ShapeDtypeStruct((_B, 4), f32),
    )(pos, batch3)

    md, idxo = pl.pallas_call(
        _argmin_body,
        grid=(nchunks,),
        in_specs=[row_spec(3), batch_spec, full((_B, 4))],
        out_specs=(full((1, _B)), full((1, _B))),
        out_shape=(jax.ShapeDtypeStruct((1, _B), f32),
                   jax.ShapeDtypeStruct((1, _B), jnp.int32)),
    )(pos, batch3, stats)

    idx = idxo[0]
    x_dst = x[idx]
    pos_dst = pos[idx]
    lframes_dst = lframes[idx]

    out = pl.pallas_call(
        functools.partial(_main_body, nchunks),
        grid=(nchunks,),
        in_specs=[row_spec(d), row_spec(3), batch_spec,
                  full((_B, d)), full((_B, 3)), full((_B, 3, 3)),
                  full((d, h)), full((d, h)), full((3, h)),
                  full((1, h)), full((h, d)), full((1, d))],
        out_specs=full((_B, d)),
        out_shape=jax.ShapeDtypeStruct((_B, d), f32),
        scratch_shapes=[pltpu.VMEM((_B, 4 * h), f32)],
    )(x, pos, batch3, x_dst, pos_dst, lframes_dst,
      W1[:d], W1[d:2 * d], W1[2 * d:], b1[None, :], W2, b2[None, :])

    return (out, pos_dst, batch[idx], lframes_dst)
